# baseline (speedup 1.0000x reference)
import functools

import jax
import jax.numpy as jnp
from jax.experimental import pallas as pl
from jax.experimental.pallas import tpu as pltpu

_INTERPRET = False

NE = 8
NT = 2048
D = 1024
F = 2816
K = 2
NA = NT * K
M = 128
NB = NA // M
NS = NB + NE - 1
NF = 2
FC = F // NF
LANES = 128
MT = 256
NTB = NT // MT


def _router_body(x_ref, rw_ref, logits_ref, probs_ref, route_ref, aux_ref):
    x = x_ref[...]
    rw = rw_ref[...]
    logits = jax.lax.dot_general(x, rw, (((1,), (1,)), ((), ())),
                                 preferred_element_type=jnp.float32)
    logits_ref[...] = logits
    cols = jax.lax.broadcasted_iota(jnp.int32, (NT, LANES), 1)
    valid = cols < NE
    neg = jnp.where(valid, logits, -jnp.inf)
    m = jnp.max(neg, axis=1, keepdims=True)
    ex = jnp.exp(neg - m)
    probs = ex / jnp.sum(ex, axis=1, keepdims=True)
    probs_ref[...] = probs
    p1 = jnp.max(jnp.where(valid, probs, -1.0), axis=1, keepdims=True)
    i1 = jnp.min(jnp.where(valid & (probs == p1), cols, LANES), axis=1,
                 keepdims=True)
    m1 = cols == i1
    p2 = jnp.max(jnp.where(valid & ~m1, probs, -1.0), axis=1, keepdims=True)
    i2 = jnp.min(jnp.where(valid & ~m1 & (probs == p2), cols, LANES), axis=1,
                 keepdims=True)
    m2 = cols == i2
    tot = p1 + p2
    w1 = p1 / tot
    w2 = p2 / tot

    oh1 = m1.astype(jnp.float32)
    oh2 = m2.astype(jnp.float32)
    tot1 = jnp.sum(oh1, axis=0, keepdims=True)
    tot2 = jnp.sum(oh2, axis=0, keepdims=True)
    counts = tot1 + tot2
    eu_r = jax.lax.broadcasted_iota(jnp.int32, (LANES, LANES), 0)
    eu_c = jax.lax.broadcasted_iota(jnp.int32, (LANES, LANES), 1)
    tri_e = (eu_r < eu_c).astype(jnp.float32)
    off_row = jax.lax.dot_general(counts, tri_e, (((1,), (0,)), ((), ())),
                                  precision=jax.lax.Precision.HIGHEST,
                                  preferred_element_type=jnp.float32)
    r_io = jax.lax.broadcasted_iota(jnp.int32, (NT, NT), 0)
    c_io = jax.lax.broadcasted_iota(jnp.int32, (NT, NT), 1)
    tri = (c_io < r_io).astype(jnp.float32)
    cnt1 = jax.lax.dot_general(tri, oh1, (((1,), (0,)), ((), ())),
                               preferred_element_type=jnp.float32)
    cnt2 = jax.lax.dot_general(tri, oh2, (((1,), (0,)), ((), ())),
                               preferred_element_type=jnp.float32)
    pos1 = jnp.sum(jnp.where(m1, off_row + cnt1, 0.0), axis=1, keepdims=True)
    pos2 = jnp.sum(jnp.where(m2, off_row + tot1 + cnt2, 0.0), axis=1,
                   keepdims=True)

    route = (jnp.where(cols == 0, i1.astype(jnp.float32), 0.0)
             + jnp.where(cols == 1, i2.astype(jnp.float32), 0.0)
             + jnp.where(cols == 2, w1, 0.0)
             + jnp.where(cols == 3, w2, 0.0)
             + jnp.where(cols == 4, pos1, 0.0)
             + jnp.where(cols == 5, pos2, 0.0))
    route_ref[...] = route
    avgp = jnp.mean(probs, axis=0, keepdims=True)
    aux_ref[...] = jnp.concatenate([counts / float(NA), avgp], axis=0)


def _run_router(x, rw_pad):
    return pl.pallas_call(
        _router_body,
        out_shape=(
            jax.ShapeDtypeStruct((NT, LANES), jnp.float32),
            jax.ShapeDtypeStruct((NT, LANES), jnp.float32),
            jax.ShapeDtypeStruct((NT, LANES), jnp.float32),
            jax.ShapeDtypeStruct((2, LANES), jnp.float32),
        ),
        compiler_params=pltpu.CompilerParams(
            vmem_limit_bytes=100 * 1024 * 1024),
        interpret=_INTERPRET,
    )(x, rw_pad)


def _gather_body(pos_ref, wts_ref, x_ref, out_ref, ws_ref):
    i = pl.program_id(0)
    base = i * M
    rows = jax.lax.broadcasted_iota(jnp.int32, (M, NT), 0) + base
    oh1 = (rows == pos_ref[0:1, :]).astype(jnp.float32)
    oh2 = (rows == pos_ref[1:2, :]).astype(jnp.float32)
    oh = oh1 + oh2
    out_ref[...] = jax.lax.dot_general(oh, x_ref[...], (((1,), (0,)), ((), ())),
                                       preferred_element_type=jnp.float32)
    ws_ref[...] = (
        jax.lax.dot_general(oh1, wts_ref[0:1, :], (((1,), (1,)), ((), ())),
                            precision=jax.lax.Precision.HIGHEST,
                            preferred_element_type=jnp.float32)
        + jax.lax.dot_general(oh2, wts_ref[1:2, :], (((1,), (1,)), ((), ())),
                              precision=jax.lax.Precision.HIGHEST,
                              preferred_element_type=jnp.float32))


def _run_gather(pos, wts, x):
    return pl.pallas_call(
        _gather_body,
        grid=(NB,),
        in_specs=[
            pl.BlockSpec((2, NT), lambda i: (0, 0)),
            pl.BlockSpec((2, NT), lambda i: (0, 0)),
            pl.BlockSpec((NT, D), lambda i: (0, 0)),
        ],
        out_specs=[
            pl.BlockSpec((M, D), lambda i: (i, 0)),
            pl.BlockSpec((M, 1), lambda i: (i, 0)),
        ],
        out_shape=[
            jax.ShapeDtypeStruct((NA, D), jnp.float32),
            jax.ShapeDtypeStruct((NA, 1), jnp.float32),
        ],
        interpret=_INTERPRET,
    )(pos, wts, x)


def _ffn_body(g_sr, b_sr, first_sr, lo_sr, hi_sr,
              xs_ref, w_ref, gw_ref, uw_ref, dw_ref, out_ref, acc_ref):
    c = pl.program_id(0)
    s = pl.program_id(1)
    lo = lo_sr[s]
    hi = hi_sr[s]
    b = b_sr[s]
    xb = xs_ref[...]
    g_ = jax.lax.dot_general(xb, gw_ref[0], (((1,), (1,)), ((), ())),
                             preferred_element_type=jnp.float32)
    u_ = jax.lax.dot_general(xb, uw_ref[0], (((1,), (1,)), ((), ())),
                             preferred_element_type=jnp.float32)
    h = g_ / (1.0 + jnp.exp(-g_)) * u_
    rows = jax.lax.broadcasted_iota(jnp.int32, (M, 1), 0)
    scale = jnp.where((rows >= lo) & (rows < hi), w_ref[...], 0.0)
    h = h * scale
    y = jax.lax.dot_general(h, dw_ref[0], (((1,), (1,)), ((), ())),
                            preferred_element_type=jnp.float32)

    fresh = (c == 0) & (first_sr[s] == 1)

    @pl.when(fresh)
    def _():
        acc_ref[b] = y

    @pl.when(~fresh)
    def _():
        acc_ref[b] += y

    @pl.when(c == NF - 1)
    def _():
        out_ref[...] = acc_ref[b]


def _run_ffn(steps, x_sorted, w_sorted, gate_w, up_w, down_w):
    g_s, b_s, first_s, lo_s, hi_s = steps
    grid_spec = pltpu.PrefetchScalarGridSpec(
        num_scalar_prefetch=5,
        grid=(NF, NS),
        in_specs=[
            pl.BlockSpec((M, D), lambda c, s, g, b, f, lo, hi: (b[s], 0)),
            pl.BlockSpec((M, 1), lambda c, s, g, b, f, lo, hi: (b[s], 0)),
            pl.BlockSpec((1, FC, D), lambda c, s, g, b, f, lo, hi: (g[s], c, 0)),
            pl.BlockSpec((1, FC, D), lambda c, s, g, b, f, lo, hi: (g[s], c, 0)),
            pl.BlockSpec((1, D, FC), lambda c, s, g, b, f, lo, hi: (g[s], 0, c)),
        ],
        out_specs=pl.BlockSpec((M, D), lambda c, s, g, b, f, lo, hi: (b[s], 0)),
        scratch_shapes=[pltpu.VMEM((NB, M, D), jnp.float32)],
    )
    return pl.pallas_call(
        _ffn_body,
        grid_spec=grid_spec,
        out_shape=jax.ShapeDtypeStruct((NA, D), jnp.float32),
        compiler_params=pltpu.CompilerParams(
            vmem_limit_bytes=128 * 1024 * 1024),
        interpret=_INTERPRET,
    )(g_s, b_s, first_s, lo_s, hi_s,
      x_sorted, w_sorted, gate_w, up_w, down_w)


def _combine_body(posT_ref, ys_ref, out_ref):
    seg = posT_ref[...]
    p1 = seg[:, 0:1]
    p2 = seg[:, 1:2]
    cols = jax.lax.broadcasted_iota(jnp.int32, (MT, NA), 1)
    oh = (cols == p1).astype(jnp.float32) + (cols == p2).astype(jnp.float32)
    out_ref[...] = jax.lax.dot_general(oh, ys_ref[...], (((1,), (0,)), ((), ())),
                                       preferred_element_type=jnp.float32)


def _run_combine(posT, y_sorted):
    return pl.pallas_call(
        _combine_body,
        grid=(NTB,),
        in_specs=[
            pl.BlockSpec((MT, 2), lambda i: (i, 0)),
            pl.BlockSpec((NA, D), lambda i: (0, 0)),
        ],
        out_specs=pl.BlockSpec((MT, D), lambda i: (i, 0)),
        out_shape=jax.ShapeDtypeStruct((NT, D), jnp.float32),
        interpret=_INTERPRET,
    )(posT, y_sorted)


def kernel(hidden_states, router_w, gate_w, up_w, down_w):
    b, s, d = hidden_states.shape
    x = hidden_states.reshape(NT, D)
    rw_pad = jnp.zeros((LANES, D), jnp.float32).at[:NE].set(router_w)

    logits_p, probs_p, route, aux = _run_router(x, rw_pad)
    logits = logits_p[:, :NE]
    probs = probs_p[:, :NE]
    expert_frac = aux[0, :NE]
    avg_prob = aux[1, :NE]

    pos = jnp.stack([route[:, 4], route[:, 5]]).astype(jnp.int32)
    wts = jnp.stack([route[:, 2], route[:, 3]])

    counts = (expert_frac * float(NA)).astype(jnp.int32)
    off = jnp.concatenate([jnp.zeros((1,), jnp.int32),
                           jnp.cumsum(counts).astype(jnp.int32)])

    b_lo = off[:NE] // M
    b_hi = (off[1:] - 1) // M
    nb = jnp.where(counts > 0, b_hi - b_lo + 1, 0)
    cnb = jnp.concatenate([jnp.zeros((1,), jnp.int32),
                           jnp.cumsum(nb).astype(jnp.int32)])
    s_ar = jnp.arange(NS, dtype=jnp.int32)
    g_s = jnp.clip(jnp.searchsorted(cnb, s_ar, side='right') - 1,
                   0, NE - 1).astype(jnp.int32)
    valid_s = s_ar < cnb[NE]
    b_s = jnp.where(valid_s, b_lo[g_s] + s_ar - cnb[g_s],
                    NB - 1).astype(jnp.int32)
    lo_s = jnp.clip(off[g_s] - b_s * M, 0, M).astype(jnp.int32)
    hi_s = jnp.where(valid_s,
                     jnp.clip(off[g_s + 1] - b_s * M, 0, M), 0).astype(jnp.int32)
    first_s = (jnp.concatenate([jnp.ones((1,), jnp.bool_),
                                b_s[1:] != b_s[:-1]])
               & valid_s).astype(jnp.int32)

    x_sorted, w_sorted = _run_gather(pos, wts, x)
    y_sorted = _run_ffn((g_s, b_s, first_s, lo_s, hi_s),
                        x_sorted, w_sorted, gate_w, up_w, down_w)
    out = _run_combine(pos.T, y_sorted)

    return (out.reshape(b, s, d), expert_frac, avg_prob, logits, probs)


# speedup vs baseline: 1.4211x; 1.4211x over previous
import functools

import jax
import jax.numpy as jnp
from jax.experimental import pallas as pl
from jax.experimental.pallas import tpu as pltpu

_INTERPRET = False

NE = 8
NT = 2048
D = 1024
F = 2816
K = 2
NA = NT * K
M = 512
NB = NA // M
NS = NB + NE - 1
NF = 2
FC = F // NF
LANES = 128
MT = 256
NTB = NT // MT


def _router_body(x_ref, rw_ref, logits_ref, probs_ref, route_ref, aux_ref):
    x = x_ref[...]
    rw = rw_ref[...]
    logits = jax.lax.dot_general(x, rw, (((1,), (1,)), ((), ())),
                                 preferred_element_type=jnp.float32)
    logits_ref[...] = logits
    cols = jax.lax.broadcasted_iota(jnp.int32, (NT, LANES), 1)
    valid = cols < NE
    neg = jnp.where(valid, logits, -jnp.inf)
    m = jnp.max(neg, axis=1, keepdims=True)
    ex = jnp.exp(neg - m)
    probs = ex / jnp.sum(ex, axis=1, keepdims=True)
    probs_ref[...] = probs
    p1 = jnp.max(jnp.where(valid, probs, -1.0), axis=1, keepdims=True)
    i1 = jnp.min(jnp.where(valid & (probs == p1), cols, LANES), axis=1,
                 keepdims=True)
    m1 = cols == i1
    p2 = jnp.max(jnp.where(valid & ~m1, probs, -1.0), axis=1, keepdims=True)
    i2 = jnp.min(jnp.where(valid & ~m1 & (probs == p2), cols, LANES), axis=1,
                 keepdims=True)
    m2 = cols == i2
    tot = p1 + p2
    w1 = p1 / tot
    w2 = p2 / tot

    oh1 = m1.astype(jnp.float32)
    oh2 = m2.astype(jnp.float32)
    tot1 = jnp.sum(oh1, axis=0, keepdims=True)
    tot2 = jnp.sum(oh2, axis=0, keepdims=True)
    counts = tot1 + tot2
    eu_r = jax.lax.broadcasted_iota(jnp.int32, (LANES, LANES), 0)
    eu_c = jax.lax.broadcasted_iota(jnp.int32, (LANES, LANES), 1)
    tri_e = (eu_r < eu_c).astype(jnp.float32)
    off_row = jax.lax.dot_general(counts, tri_e, (((1,), (0,)), ((), ())),
                                  precision=jax.lax.Precision.HIGHEST,
                                  preferred_element_type=jnp.float32)
    r_io = jax.lax.broadcasted_iota(jnp.int32, (NT, NT), 0)
    c_io = jax.lax.broadcasted_iota(jnp.int32, (NT, NT), 1)
    tri = (c_io < r_io).astype(jnp.float32)
    cnt1 = jax.lax.dot_general(tri, oh1, (((1,), (0,)), ((), ())),
                               preferred_element_type=jnp.float32)
    cnt2 = jax.lax.dot_general(tri, oh2, (((1,), (0,)), ((), ())),
                               preferred_element_type=jnp.float32)
    pos1 = jnp.sum(jnp.where(m1, off_row + cnt1, 0.0), axis=1, keepdims=True)
    pos2 = jnp.sum(jnp.where(m2, off_row + tot1 + cnt2, 0.0), axis=1,
                   keepdims=True)

    route = (jnp.where(cols == 0, i1.astype(jnp.float32), 0.0)
             + jnp.where(cols == 1, i2.astype(jnp.float32), 0.0)
             + jnp.where(cols == 2, w1, 0.0)
             + jnp.where(cols == 3, w2, 0.0)
             + jnp.where(cols == 4, pos1, 0.0)
             + jnp.where(cols == 5, pos2, 0.0))
    route_ref[...] = route
    avgp = jnp.mean(probs, axis=0, keepdims=True)
    aux_ref[...] = jnp.concatenate([counts / float(NA), avgp], axis=0)


def _run_router(x, rw_pad):
    return pl.pallas_call(
        _router_body,
        out_shape=(
            jax.ShapeDtypeStruct((NT, LANES), jnp.float32),
            jax.ShapeDtypeStruct((NT, LANES), jnp.float32),
            jax.ShapeDtypeStruct((NT, LANES), jnp.float32),
            jax.ShapeDtypeStruct((2, LANES), jnp.float32),
        ),
        compiler_params=pltpu.CompilerParams(
            vmem_limit_bytes=100 * 1024 * 1024),
        interpret=_INTERPRET,
    )(x, rw_pad)


def _gather_body(pos_ref, wts_ref, x_ref, out_ref, ws_ref):
    i = pl.program_id(0)
    base = i * M
    rows = jax.lax.broadcasted_iota(jnp.int32, (M, NT), 0) + base
    oh1 = (rows == pos_ref[0:1, :]).astype(jnp.float32)
    oh2 = (rows == pos_ref[1:2, :]).astype(jnp.float32)
    oh = oh1 + oh2
    out_ref[...] = jax.lax.dot_general(oh, x_ref[...], (((1,), (0,)), ((), ())),
                                       preferred_element_type=jnp.float32)
    ws_ref[...] = (
        jax.lax.dot_general(oh1, wts_ref[0:1, :], (((1,), (1,)), ((), ())),
                            precision=jax.lax.Precision.HIGHEST,
                            preferred_element_type=jnp.float32)
        + jax.lax.dot_general(oh2, wts_ref[1:2, :], (((1,), (1,)), ((), ())),
                              precision=jax.lax.Precision.HIGHEST,
                              preferred_element_type=jnp.float32))


def _run_gather(pos, wts, x):
    return pl.pallas_call(
        _gather_body,
        grid=(NB,),
        in_specs=[
            pl.BlockSpec((2, NT), lambda i: (0, 0)),
            pl.BlockSpec((2, NT), lambda i: (0, 0)),
            pl.BlockSpec((NT, D), lambda i: (0, 0)),
        ],
        out_specs=[
            pl.BlockSpec((M, D), lambda i: (i, 0)),
            pl.BlockSpec((M, 1), lambda i: (i, 0)),
        ],
        out_shape=[
            jax.ShapeDtypeStruct((NA, D), jnp.float32),
            jax.ShapeDtypeStruct((NA, 1), jnp.float32),
        ],
        interpret=_INTERPRET,
    )(pos, wts, x)


def _ffn_body(g_sr, b_sr, first_sr, lo_sr, hi_sr,
              xs_ref, w_ref, gw_ref, uw_ref, dw_ref, out_ref, acc_ref):
    c = pl.program_id(0)
    s = pl.program_id(1)
    lo = lo_sr[s]
    hi = hi_sr[s]
    b = b_sr[s]
    xb = xs_ref[...]
    g_ = jax.lax.dot_general(xb, gw_ref[0], (((1,), (1,)), ((), ())),
                             preferred_element_type=jnp.float32)
    u_ = jax.lax.dot_general(xb, uw_ref[0], (((1,), (1,)), ((), ())),
                             preferred_element_type=jnp.float32)
    h = g_ / (1.0 + jnp.exp(-g_)) * u_
    rows = jax.lax.broadcasted_iota(jnp.int32, (M, 1), 0)
    scale = jnp.where((rows >= lo) & (rows < hi), w_ref[...], 0.0)
    h = h * scale
    y = jax.lax.dot_general(h, dw_ref[0], (((1,), (1,)), ((), ())),
                            preferred_element_type=jnp.float32)

    fresh = (c == 0) & (first_sr[s] == 1)

    @pl.when(fresh)
    def _():
        acc_ref[b] = y

    @pl.when(~fresh)
    def _():
        acc_ref[b] += y

    @pl.when(c == NF - 1)
    def _():
        out_ref[...] = acc_ref[b]


def _run_ffn(steps, x_sorted, w_sorted, gate_w, up_w, down_w):
    g_s, b_s, first_s, lo_s, hi_s = steps
    grid_spec = pltpu.PrefetchScalarGridSpec(
        num_scalar_prefetch=5,
        grid=(NF, NS),
        in_specs=[
            pl.BlockSpec((M, D), lambda c, s, g, b, f, lo, hi: (b[s], 0)),
            pl.BlockSpec((M, 1), lambda c, s, g, b, f, lo, hi: (b[s], 0)),
            pl.BlockSpec((1, FC, D), lambda c, s, g, b, f, lo, hi: (g[s], c, 0)),
            pl.BlockSpec((1, FC, D), lambda c, s, g, b, f, lo, hi: (g[s], c, 0)),
            pl.BlockSpec((1, D, FC), lambda c, s, g, b, f, lo, hi: (g[s], 0, c)),
        ],
        out_specs=pl.BlockSpec((M, D), lambda c, s, g, b, f, lo, hi: (b[s], 0)),
        scratch_shapes=[pltpu.VMEM((NB, M, D), jnp.float32)],
    )
    return pl.pallas_call(
        _ffn_body,
        grid_spec=grid_spec,
        out_shape=jax.ShapeDtypeStruct((NA, D), jnp.float32),
        compiler_params=pltpu.CompilerParams(
            vmem_limit_bytes=128 * 1024 * 1024),
        interpret=_INTERPRET,
    )(g_s, b_s, first_s, lo_s, hi_s,
      x_sorted, w_sorted, gate_w, up_w, down_w)


def _combine_body(posT_ref, ys_ref, out_ref):
    seg = posT_ref[...]
    p1 = seg[:, 0:1]
    p2 = seg[:, 1:2]
    cols = jax.lax.broadcasted_iota(jnp.int32, (MT, NA), 1)
    oh = (cols == p1).astype(jnp.float32) + (cols == p2).astype(jnp.float32)
    out_ref[...] = jax.lax.dot_general(oh, ys_ref[...], (((1,), (0,)), ((), ())),
                                       preferred_element_type=jnp.float32)


def _run_combine(posT, y_sorted):
    return pl.pallas_call(
        _combine_body,
        grid=(NTB,),
        in_specs=[
            pl.BlockSpec((MT, 2), lambda i: (i, 0)),
            pl.BlockSpec((NA, D), lambda i: (0, 0)),
        ],
        out_specs=pl.BlockSpec((MT, D), lambda i: (i, 0)),
        out_shape=jax.ShapeDtypeStruct((NT, D), jnp.float32),
        interpret=_INTERPRET,
    )(posT, y_sorted)


def kernel(hidden_states, router_w, gate_w, up_w, down_w):
    b, s, d = hidden_states.shape
    x = hidden_states.reshape(NT, D)
    rw_pad = jnp.zeros((LANES, D), jnp.float32).at[:NE].set(router_w)

    logits_p, probs_p, route, aux = _run_router(x, rw_pad)
    logits = logits_p[:, :NE]
    probs = probs_p[:, :NE]
    expert_frac = aux[0, :NE]
    avg_prob = aux[1, :NE]

    pos = jnp.stack([route[:, 4], route[:, 5]]).astype(jnp.int32)
    wts = jnp.stack([route[:, 2], route[:, 3]])

    counts = (expert_frac * float(NA)).astype(jnp.int32)
    off = jnp.concatenate([jnp.zeros((1,), jnp.int32),
                           jnp.cumsum(counts).astype(jnp.int32)])

    b_lo = off[:NE] // M
    b_hi = (off[1:] - 1) // M
    nb = jnp.where(counts > 0, b_hi - b_lo + 1, 0)
    cnb = jnp.concatenate([jnp.zeros((1,), jnp.int32),
                           jnp.cumsum(nb).astype(jnp.int32)])
    s_ar = jnp.arange(NS, dtype=jnp.int32)
    g_s = jnp.clip(jnp.searchsorted(cnb, s_ar, side='right') - 1,
                   0, NE - 1).astype(jnp.int32)
    valid_s = s_ar < cnb[NE]
    b_s = jnp.where(valid_s, b_lo[g_s] + s_ar - cnb[g_s],
                    NB - 1).astype(jnp.int32)
    lo_s = jnp.clip(off[g_s] - b_s * M, 0, M).astype(jnp.int32)
    hi_s = jnp.where(valid_s,
                     jnp.clip(off[g_s + 1] - b_s * M, 0, M), 0).astype(jnp.int32)
    first_s = (jnp.concatenate([jnp.ones((1,), jnp.bool_),
                                b_s[1:] != b_s[:-1]])
               & valid_s).astype(jnp.int32)

    x_sorted, w_sorted = _run_gather(pos, wts, x)
    y_sorted = _run_ffn((g_s, b_s, first_s, lo_s, hi_s),
                        x_sorted, w_sorted, gate_w, up_w, down_w)
    out = _run_combine(pos.T, y_sorted)

    return (out.reshape(b, s, d), expert_frac, avg_prob, logits, probs)


# speedup vs baseline: 1.5392x; 1.0831x over previous
import functools

import jax
import jax.numpy as jnp
from jax import lax
from jax.experimental import pallas as pl
from jax.experimental.pallas import tpu as pltpu
from jax.experimental.pallas import tpu_sc as plsc

_INTERPRET = False

NE = 8
NT = 2048
D = 1024
F = 2816
K = 2
NA = NT * K
M = 512
NB = NA // M
NS = NB + NE - 1
NF = 2
FC = F // NF
LANES = 128
MT = 256
NTB = NT // MT


def _router_body(x_ref, rw_ref, logits_ref, probs_ref, route_ref, aux_ref):
    x = x_ref[...]
    rw = rw_ref[...]
    logits = jax.lax.dot_general(x, rw, (((1,), (1,)), ((), ())),
                                 preferred_element_type=jnp.float32)
    logits_ref[...] = logits
    cols = jax.lax.broadcasted_iota(jnp.int32, (NT, LANES), 1)
    valid = cols < NE
    neg = jnp.where(valid, logits, -jnp.inf)
    m = jnp.max(neg, axis=1, keepdims=True)
    ex = jnp.exp(neg - m)
    probs = ex / jnp.sum(ex, axis=1, keepdims=True)
    probs_ref[...] = probs
    p1 = jnp.max(jnp.where(valid, probs, -1.0), axis=1, keepdims=True)
    i1 = jnp.min(jnp.where(valid & (probs == p1), cols, LANES), axis=1,
                 keepdims=True)
    m1 = cols == i1
    p2 = jnp.max(jnp.where(valid & ~m1, probs, -1.0), axis=1, keepdims=True)
    i2 = jnp.min(jnp.where(valid & ~m1 & (probs == p2), cols, LANES), axis=1,
                 keepdims=True)
    m2 = cols == i2
    tot = p1 + p2
    w1 = p1 / tot
    w2 = p2 / tot

    oh1 = m1.astype(jnp.float32)
    oh2 = m2.astype(jnp.float32)
    tot1 = jnp.sum(oh1, axis=0, keepdims=True)
    tot2 = jnp.sum(oh2, axis=0, keepdims=True)
    counts = tot1 + tot2
    eu_r = jax.lax.broadcasted_iota(jnp.int32, (LANES, LANES), 0)
    eu_c = jax.lax.broadcasted_iota(jnp.int32, (LANES, LANES), 1)
    tri_e = (eu_r < eu_c).astype(jnp.float32)
    off_row = jax.lax.dot_general(counts, tri_e, (((1,), (0,)), ((), ())),
                                  precision=jax.lax.Precision.HIGHEST,
                                  preferred_element_type=jnp.float32)
    r_io = jax.lax.broadcasted_iota(jnp.int32, (NT, NT), 0)
    c_io = jax.lax.broadcasted_iota(jnp.int32, (NT, NT), 1)
    tri = (c_io < r_io).astype(jnp.float32)
    cnt1 = jax.lax.dot_general(tri, oh1, (((1,), (0,)), ((), ())),
                               preferred_element_type=jnp.float32)
    cnt2 = jax.lax.dot_general(tri, oh2, (((1,), (0,)), ((), ())),
                               preferred_element_type=jnp.float32)
    pos1 = jnp.sum(jnp.where(m1, off_row + cnt1, 0.0), axis=1, keepdims=True)
    pos2 = jnp.sum(jnp.where(m2, off_row + tot1 + cnt2, 0.0), axis=1,
                   keepdims=True)

    route = (jnp.where(cols == 0, i1.astype(jnp.float32), 0.0)
             + jnp.where(cols == 1, i2.astype(jnp.float32), 0.0)
             + jnp.where(cols == 2, w1, 0.0)
             + jnp.where(cols == 3, w2, 0.0)
             + jnp.where(cols == 4, pos1, 0.0)
             + jnp.where(cols == 5, pos2, 0.0))
    route_ref[...] = route
    avgp = jnp.mean(probs, axis=0, keepdims=True)
    aux_ref[...] = jnp.concatenate([counts / float(NA), avgp], axis=0)


def _run_router(x, rw_pad):
    return pl.pallas_call(
        _router_body,
        out_shape=(
            jax.ShapeDtypeStruct((NT, LANES), jnp.float32),
            jax.ShapeDtypeStruct((NT, LANES), jnp.float32),
            jax.ShapeDtypeStruct((NT, LANES), jnp.float32),
            jax.ShapeDtypeStruct((2, LANES), jnp.float32),
        ),
        compiler_params=pltpu.CompilerParams(
            vmem_limit_bytes=100 * 1024 * 1024),
        interpret=_INTERPRET,
    )(x, rw_pad)


def _gather_body(pos_ref, wts_ref, x_ref, out_ref, ws_ref):
    i = pl.program_id(0)
    base = i * M
    rows = jax.lax.broadcasted_iota(jnp.int32, (M, NT), 0) + base
    oh1 = (rows == pos_ref[0:1, :]).astype(jnp.float32)
    oh2 = (rows == pos_ref[1:2, :]).astype(jnp.float32)
    oh = oh1 + oh2
    out_ref[...] = jax.lax.dot_general(oh, x_ref[...], (((1,), (0,)), ((), ())),
                                       preferred_element_type=jnp.float32)
    ws_ref[...] = (
        jax.lax.dot_general(oh1, wts_ref[0:1, :], (((1,), (1,)), ((), ())),
                            precision=jax.lax.Precision.HIGHEST,
                            preferred_element_type=jnp.float32)
        + jax.lax.dot_general(oh2, wts_ref[1:2, :], (((1,), (1,)), ((), ())),
                              precision=jax.lax.Precision.HIGHEST,
                              preferred_element_type=jnp.float32))


def _run_gather(pos, wts, x):
    return pl.pallas_call(
        _gather_body,
        grid=(NB,),
        in_specs=[
            pl.BlockSpec((2, NT), lambda i: (0, 0)),
            pl.BlockSpec((2, NT), lambda i: (0, 0)),
            pl.BlockSpec((NT, D), lambda i: (0, 0)),
        ],
        out_specs=[
            pl.BlockSpec((M, D), lambda i: (i, 0)),
            pl.BlockSpec((M, 1), lambda i: (i, 0)),
        ],
        out_shape=[
            jax.ShapeDtypeStruct((NA, D), jnp.float32),
            jax.ShapeDtypeStruct((NA, 1), jnp.float32),
        ],
        interpret=_INTERPRET,
    )(pos, wts, x)


NC = 2
NSUB = 16
NW = NC * NSUB
TPW = NT // NW
SUB = 32


def _sc_dispatch_body(x_hbm, pos_hbm, w_hbm, xs_hbm, ws_hbm,
                      rows_v, idx_v, w_v, sem):
    wid = lax.axis_index("s") * NC + lax.axis_index("c")
    pltpu.sync_copy(x_hbm.at[pl.ds(wid * TPW, TPW)], rows_v)
    pltpu.sync_copy(pos_hbm.at[wid], idx_v)
    pltpu.sync_copy(w_hbm.at[wid], w_v)
    for k in range(K):
        pltpu.async_copy(rows_v, xs_hbm.at[idx_v.at[k]], sem).wait()
        pltpu.async_copy(w_v.at[k], ws_hbm.at[idx_v.at[k]], sem).wait()


def _run_sc_dispatch(x, pos_w, wts_w):
    f = functools.partial(
        pl.kernel,
        out_type=(
            jax.ShapeDtypeStruct((NA, D), jnp.float32),
            jax.ShapeDtypeStruct((NA, LANES), jnp.float32),
        ),
        mesh=plsc.VectorSubcoreMesh(core_axis_name="c", subcore_axis_name="s"),
        scratch_types=(
            pltpu.VMEM((TPW, D), jnp.float32),
            pltpu.VMEM((K, TPW), jnp.int32),
            pltpu.VMEM((K, TPW, LANES), jnp.float32),
            pltpu.SemaphoreType.DMA,
        ),
    )(_sc_dispatch_body)
    return f(x, pos_w, wts_w)


def _sc_combine_body(y_hbm, pos_hbm, out_hbm, idx_v, a_v, b_v, sem):
    wid = lax.axis_index("s") * NC + lax.axis_index("c")
    pltpu.sync_copy(pos_hbm.at[wid], idx_v)
    for sub in range(TPW // SUB):
        pltpu.async_copy(y_hbm.at[idx_v.at[0, pl.ds(sub * SUB, SUB)]],
                         a_v, sem).wait()
        pltpu.async_copy(y_hbm.at[idx_v.at[1, pl.ds(sub * SUB, SUB)]],
                         b_v, sem).wait()

        def _add_row(r, carry):
            for u in range(D // 16):
                a_v[r, pl.ds(u * 16, 16)] += b_v[r, pl.ds(u * 16, 16)]
            return carry

        lax.fori_loop(0, SUB, _add_row, 0)
        pltpu.sync_copy(a_v, out_hbm.at[pl.ds(wid * TPW + sub * SUB, SUB)])


def _run_sc_combine(y_sorted, pos_w):
    f = functools.partial(
        pl.kernel,
        out_type=jax.ShapeDtypeStruct((NT, D), jnp.float32),
        mesh=plsc.VectorSubcoreMesh(core_axis_name="c", subcore_axis_name="s"),
        scratch_types=(
            pltpu.VMEM((K, TPW), jnp.int32),
            pltpu.VMEM((SUB, D), jnp.float32),
            pltpu.VMEM((SUB, D), jnp.float32),
            pltpu.SemaphoreType.DMA,
        ),
    )(_sc_combine_body)
    return f(y_sorted, pos_w)


def _ffn_body(g_sr, b_sr, first_sr, lo_sr, hi_sr,
              xs_ref, w_ref, gw_ref, uw_ref, dw_ref, out_ref, acc_ref):
    c = pl.program_id(0)
    s = pl.program_id(1)
    lo = lo_sr[s]
    hi = hi_sr[s]
    b = b_sr[s]
    xb = xs_ref[...]
    g_ = jax.lax.dot_general(xb, gw_ref[0], (((1,), (1,)), ((), ())),
                             preferred_element_type=jnp.float32)
    u_ = jax.lax.dot_general(xb, uw_ref[0], (((1,), (1,)), ((), ())),
                             preferred_element_type=jnp.float32)
    h = g_ / (1.0 + jnp.exp(-g_)) * u_
    rows = jax.lax.broadcasted_iota(jnp.int32, (M, 1), 0)
    scale = jnp.where((rows >= lo) & (rows < hi), w_ref[:, 0:1], 0.0)
    h = h * scale
    y = jax.lax.dot_general(h, dw_ref[0], (((1,), (1,)), ((), ())),
                            preferred_element_type=jnp.float32)

    fresh = (c == 0) & (first_sr[s] == 1)

    @pl.when(fresh)
    def _():
        acc_ref[b] = y

    @pl.when(~fresh)
    def _():
        acc_ref[b] += y

    @pl.when(c == NF - 1)
    def _():
        out_ref[...] = acc_ref[b]


def _run_ffn(steps, x_sorted, w_sorted, gate_w, up_w, down_w):
    g_s, b_s, first_s, lo_s, hi_s = steps
    grid_spec = pltpu.PrefetchScalarGridSpec(
        num_scalar_prefetch=5,
        grid=(NF, NS),
        in_specs=[
            pl.BlockSpec((M, D), lambda c, s, g, b, f, lo, hi: (b[s], 0)),
            pl.BlockSpec((M, LANES), lambda c, s, g, b, f, lo, hi: (b[s], 0)),
            pl.BlockSpec((1, FC, D), lambda c, s, g, b, f, lo, hi: (g[s], c, 0)),
            pl.BlockSpec((1, FC, D), lambda c, s, g, b, f, lo, hi: (g[s], c, 0)),
            pl.BlockSpec((1, D, FC), lambda c, s, g, b, f, lo, hi: (g[s], 0, c)),
        ],
        out_specs=pl.BlockSpec((M, D), lambda c, s, g, b, f, lo, hi: (b[s], 0)),
        scratch_shapes=[pltpu.VMEM((NB, M, D), jnp.float32)],
    )
    return pl.pallas_call(
        _ffn_body,
        grid_spec=grid_spec,
        out_shape=jax.ShapeDtypeStruct((NA, D), jnp.float32),
        compiler_params=pltpu.CompilerParams(
            vmem_limit_bytes=128 * 1024 * 1024),
        interpret=_INTERPRET,
    )(g_s, b_s, first_s, lo_s, hi_s,
      x_sorted, w_sorted, gate_w, up_w, down_w)


def _combine_body(posT_ref, ys_ref, out_ref):
    seg = posT_ref[...]
    p1 = seg[:, 0:1]
    p2 = seg[:, 1:2]
    cols = jax.lax.broadcasted_iota(jnp.int32, (MT, NA), 1)
    oh = (cols == p1).astype(jnp.float32) + (cols == p2).astype(jnp.float32)
    out_ref[...] = jax.lax.dot_general(oh, ys_ref[...], (((1,), (0,)), ((), ())),
                                       preferred_element_type=jnp.float32)


def _run_combine(posT, y_sorted):
    return pl.pallas_call(
        _combine_body,
        grid=(NTB,),
        in_specs=[
            pl.BlockSpec((MT, 2), lambda i: (i, 0)),
            pl.BlockSpec((NA, D), lambda i: (0, 0)),
        ],
        out_specs=pl.BlockSpec((MT, D), lambda i: (i, 0)),
        out_shape=jax.ShapeDtypeStruct((NT, D), jnp.float32),
        interpret=_INTERPRET,
    )(posT, y_sorted)


def kernel(hidden_states, router_w, gate_w, up_w, down_w):
    b, s, d = hidden_states.shape
    x = hidden_states.reshape(NT, D)
    rw_pad = jnp.zeros((LANES, D), jnp.float32).at[:NE].set(router_w)

    logits_p, probs_p, route, aux = _run_router(x, rw_pad)
    logits = logits_p[:, :NE]
    probs = probs_p[:, :NE]
    expert_frac = aux[0, :NE]
    avg_prob = aux[1, :NE]

    pos = jnp.stack([route[:, 4], route[:, 5]]).astype(jnp.int32)
    wts = jnp.stack([route[:, 2], route[:, 3]])

    counts = (expert_frac * float(NA)).astype(jnp.int32)
    off = jnp.concatenate([jnp.zeros((1,), jnp.int32),
                           jnp.cumsum(counts).astype(jnp.int32)])

    b_lo = off[:NE] // M
    b_hi = (off[1:] - 1) // M
    nb = jnp.where(counts > 0, b_hi - b_lo + 1, 0)
    cnb = jnp.concatenate([jnp.zeros((1,), jnp.int32),
                           jnp.cumsum(nb).astype(jnp.int32)])
    s_ar = jnp.arange(NS, dtype=jnp.int32)
    g_s = jnp.clip(jnp.searchsorted(cnb, s_ar, side='right') - 1,
                   0, NE - 1).astype(jnp.int32)
    valid_s = s_ar < cnb[NE]
    b_s = jnp.where(valid_s, b_lo[g_s] + s_ar - cnb[g_s],
                    NB - 1).astype(jnp.int32)
    lo_s = jnp.clip(off[g_s] - b_s * M, 0, M).astype(jnp.int32)
    hi_s = jnp.where(valid_s,
                     jnp.clip(off[g_s + 1] - b_s * M, 0, M), 0).astype(jnp.int32)
    first_s = (jnp.concatenate([jnp.ones((1,), jnp.bool_),
                                b_s[1:] != b_s[:-1]])
               & valid_s).astype(jnp.int32)

    pos_w = pos.reshape(K, NW, TPW).transpose(1, 0, 2)
    wts_w = jnp.broadcast_to(
        wts.reshape(K, NW, TPW).transpose(1, 0, 2)[..., None],
        (NW, K, TPW, LANES))

    x_sorted, w_sorted = _run_sc_dispatch(x, pos_w, wts_w)
    y_sorted = _run_ffn((g_s, b_s, first_s, lo_s, hi_s),
                        x_sorted, w_sorted, gate_w, up_w, down_w)
    out = _run_sc_combine(y_sorted, pos_w[:, :, :])

    return (out.reshape(b, s, d), expert_frac, avg_prob, logits, probs)
